# Initial kernel scaffold; baseline (speedup 1.0000x reference)
import jax
import jax.numpy as jnp
from jax.experimental import pallas as pl


def kernel(ll_edges, ll_data_matrix, dd_edges, dd_data_matrix, params):
    raise NotImplementedError("write your pallas kernel here")


# speedup vs baseline: 1.0000x; 1.0000x over previous
import jax, jax.numpy as jnp
from jax.experimental import pallas as pl
from math import sqrt

_N = 10000
_F = 128


def _gcn(x, ei, ew, p, n):
    loop = jnp.arange(n)
    row = jnp.concatenate([ei[0], loop])
    col = jnp.concatenate([ei[1], loop])
    e = jnp.concatenate([ew, jnp.ones((n,), ew.dtype)])
    deg = jax.ops.segment_sum(e, col, num_segments=n)
    dinv = jnp.where(deg > 0, 1.0 / jnp.sqrt(jnp.where(deg > 0, deg, 1.0)), 0.0)
    norm = dinv[row] * e * dinv[col]
    xw = x @ p["W"]
    return jax.ops.segment_sum(xw[row] * norm[:, None], col, num_segments=n) + p["b"]


def _gat(x, ei, ew, p, n):
    loop = jnp.arange(n)
    src = jnp.concatenate([ei[0], loop])
    dst = jnp.concatenate([ei[1], loop])
    ea = jnp.concatenate([ew, jnp.full((n,), ew.mean(), ew.dtype)])[:, None]
    xw = x @ p["W"]
    a_src = xw @ p["att_src"]
    a_dst = xw @ p["att_dst"]
    a_edge = (ea @ p["W_e"]) @ p["att_edge"]
    alpha = a_src[src] + a_dst[dst] + a_edge
    alpha = jax.nn.leaky_relu(alpha, 0.2)
    amax = jax.ops.segment_max(alpha, dst, num_segments=n)
    amax = jnp.where(jnp.isfinite(amax), amax, 0.0)
    ex = jnp.exp(alpha - amax[dst])
    denom = jax.ops.segment_sum(ex, dst, num_segments=n)
    coef = ex / (denom[dst] + 1e-16)
    return jax.ops.segment_sum(xw[src] * coef[:, None], dst, num_segments=n) + p["b"]


def _att(X, p):
    Q = X @ p["qW"].T + p["qb"]
    K = X @ p["kW"].T + p["kb"]
    V = X @ p["vW"].T + p["vb"]
    s = jnp.einsum('nld,nmd->nlm', Q, K) * (1.0 / sqrt(Q.shape[-1]))
    a = jax.nn.softmax(s, axis=1)
    z = jnp.einsum('nlm,nmd->nld', a, V)
    return jnp.einsum('nld,l->nd', z, p["cW"][0, :, 0]) + p["cb"][0]


def _branch(x, edges, dm, p1, pg, p2, n):
    ew = dm[edges[0], edges[1]]
    h1 = jax.nn.relu(_gcn(x, edges, ew, p1, n))
    ha = jax.nn.relu(_gat(h1, edges, ew, pg, n))
    h2 = jax.nn.relu(_gcn(ha, edges, ew, p2, n))
    return h1, h2


def kernel(ll_edges, ll_data_matrix, dd_edges, dd_data_matrix, params):
    kx = jax.random.key(1)
    x_lnc = jax.random.normal(jax.random.fold_in(kx, 0), (_N, _F), dtype=jnp.float32)
    x_dis = jax.random.normal(jax.random.fold_in(kx, 1), (_N, _F), dtype=jnp.float32)
    l1, l2 = _branch(x_lnc, ll_edges, ll_data_matrix, params["gcn_lnc1"], params["gat_lnc1"], params["gcn_lnc2"], _N)
    d1, d2 = _branch(x_dis, dd_edges, dd_data_matrix, params["gcn_dis1"], params["gat_dis1"], params["gcn_dis2"], _N)
    lnc = _att(jnp.stack([l1, l2], 1), params["att_lnc"])
    dis = _att(jnp.stack([d1, d2], 1), params["att_dis"])
    return lnc @ dis.T, lnc, dis


# speedup vs baseline: 9.6414x; 9.6413x over previous
import functools
from math import sqrt

import jax
import jax.numpy as jnp
from jax import lax
from jax.experimental import pallas as pl
from jax.experimental.pallas import tpu as pltpu
from jax.experimental.pallas import tpu_sc as plsc

_N = 10000
_F = 128
_NP = 10240
_C = 128
_NC = 2
_NS = 16
_NW = _NC * _NS
_BM = 1000
_GRID = _N // _BM
_SEG = _NP // _NS
_NPR = _NP // 128

_f32 = jnp.float32
_i32 = jnp.int32


def _sc_mesh():
    return plsc.VectorSubcoreMesh(core_axis_name="c", subcore_axis_name="s", num_cores=_NC, num_subcores=_NS)


_SC_PARAMS = pltpu.CompilerParams(needs_layout_passes=False)


def _make_ew_deg(E):
    nch = E // _C
    jmax = (nch + _NW - 1) // _NW

    @functools.partial(
        pl.kernel,
        out_type=(
            jax.ShapeDtypeStruct((E,), _f32),
            jax.ShapeDtypeStruct((_NC, _NP), _f32),
            jax.ShapeDtypeStruct((_NW, 16), _f32),
        ),
        mesh=_sc_mesh(),
        compiler_params=_SC_PARAMS,
        scratch_types=[
            pltpu.VMEM((_C,), _i32),
            pltpu.VMEM((_C,), _i32),
            pltpu.VMEM((_C,), _i32),
            pltpu.VMEM((_C,), _i32),
            pltpu.VMEM((_C, 128), _f32),
            pltpu.VMEM((_C,), _f32),
            pltpu.VMEM((16,), _f32),
            pltpu.VMEM_SHARED((_NP,), _f32),
            pltpu.SemaphoreType.DMA,
        ],
    )
    def k(dm128, srch, dsth, zv, ew_out, deg_out, sum_out,
          srcb, dstb, rowb, laneb, dmrows, ewb, sumb, acc, sem):
        cid = lax.axis_index("c")
        sid = lax.axis_index("s")
        wid = sid * _NC + cid
        pltpu.sync_copy(zv.at[pl.ds(sid * _SEG, _SEG)], acc.at[pl.ds(sid * _SEG, _SEG)])
        sumb[...] = jnp.zeros((16,), _f32)
        plsc.subcore_barrier()

        def body(j, carry):
            kk = j * _NW + wid

            @pl.when(kk < nch)
            def _():
                off = kk * _C
                pltpu.sync_copy(srch.at[pl.ds(off, _C)], srcb)
                pltpu.sync_copy(dsth.at[pl.ds(off, _C)], dstb)
                for g in range(_C // 16):
                    s16 = srcb[pl.ds(g * 16, 16)]
                    d16 = dstb[pl.ds(g * 16, 16)]
                    flat = s16 * _N + d16
                    rowb[pl.ds(g * 16, 16)] = lax.shift_right_logical(flat, 7)
                    laneb[pl.ds(g * 16, 16)] = lax.bitwise_and(flat, 127)
                pltpu.async_copy(dm128.at[rowb], dmrows, sem).wait()
                for g in range(_C // 16):
                    rloc = lax.iota(_i32, 16) + g * 16
                    ew16 = plsc.load_gather(dmrows, [rloc, laneb[pl.ds(g * 16, 16)]])
                    ewb[pl.ds(g * 16, 16)] = ew16
                    sumb[...] = sumb[...] + ew16
                pltpu.sync_copy(ewb, ew_out.at[pl.ds(off, _C)])
                pltpu.sync_copy(ewb, acc.at[dstb], add=True)
            return carry

        lax.fori_loop(0, jmax, body, 0)
        pltpu.sync_copy(sumb, sum_out.at[wid])
        plsc.subcore_barrier()
        pltpu.sync_copy(acc.at[pl.ds(sid * _SEG, _SEG)],
                        deg_out.at[cid, pl.ds(sid * _SEG, _SEG)])

    return k


def _make_alpha(E):
    nch = E // _C
    jmax = (nch + _NW - 1) // _NW

    @functools.partial(
        pl.kernel,
        out_type=(
            jax.ShapeDtypeStruct((E,), _f32),
            jax.ShapeDtypeStruct((_NW, 16), _f32),
        ),
        mesh=_sc_mesh(),
        compiler_params=_SC_PARAMS,
        scratch_types=[
            pltpu.VMEM((_NP,), _f32),
            pltpu.VMEM((_NP,), _f32),
            pltpu.VMEM((_C,), _i32),
            pltpu.VMEM((_C,), _i32),
            pltpu.VMEM((_C,), _f32),
            pltpu.VMEM((_C,), _f32),
            pltpu.VMEM((16,), _f32),
            pltpu.VMEM((16,), _f32),
        ],
    )
    def k(asrch, adsth, ewh, srch, dsth, cvh, al_out, max_out,
          asv, adv, srcb, dstb, ewb, alb, maxb, cvb):
        cid = lax.axis_index("c")
        sid = lax.axis_index("s")
        wid = sid * _NC + cid
        pltpu.sync_copy(asrch, asv)
        pltpu.sync_copy(adsth, adv)
        pltpu.sync_copy(cvh, cvb)
        maxb[...] = jnp.full((16,), -3.0e38, _f32)

        def body(j, carry):
            kk = j * _NW + wid

            @pl.when(kk < nch)
            def _():
                off = kk * _C
                pltpu.sync_copy(srch.at[pl.ds(off, _C)], srcb)
                pltpu.sync_copy(dsth.at[pl.ds(off, _C)], dstb)
                pltpu.sync_copy(ewh.at[pl.ds(off, _C)], ewb)
                cv = cvb[...]
                for g in range(_C // 16):
                    s16 = srcb[pl.ds(g * 16, 16)]
                    d16 = dstb[pl.ds(g * 16, 16)]
                    ew16 = ewb[pl.ds(g * 16, 16)]
                    a16 = (plsc.load_gather(asv, [s16])
                           + plsc.load_gather(adv, [d16]) + cv * ew16)
                    a16 = jnp.maximum(a16, 0.2 * a16)
                    alb[pl.ds(g * 16, 16)] = a16
                    maxb[...] = jnp.maximum(maxb[...], a16)
                pltpu.sync_copy(alb, al_out.at[pl.ds(off, _C)])
            return carry

        lax.fori_loop(0, jmax, body, 0)
        pltpu.sync_copy(maxb, max_out.at[wid])

    return k


def _make_denom(E):
    nch = E // _C
    jmax = (nch + _NW - 1) // _NW

    @functools.partial(
        pl.kernel,
        out_type=(
            jax.ShapeDtypeStruct((E,), _f32),
            jax.ShapeDtypeStruct((_NC, _NP), _f32),
        ),
        mesh=_sc_mesh(),
        compiler_params=_SC_PARAMS,
        scratch_types=[
            pltpu.VMEM((_C,), _f32),
            pltpu.VMEM((_C,), _f32),
            pltpu.VMEM((_C,), _i32),
            pltpu.VMEM((16,), _f32),
            pltpu.VMEM_SHARED((_NP,), _f32),
        ],
    )
    def k(alh, dsth, gvh, zv, ex_out, den_out, alb, exb, dstb, gvb, acc):
        cid = lax.axis_index("c")
        sid = lax.axis_index("s")
        wid = sid * _NC + cid
        pltpu.sync_copy(zv.at[pl.ds(sid * _SEG, _SEG)], acc.at[pl.ds(sid * _SEG, _SEG)])
        pltpu.sync_copy(gvh, gvb)
        plsc.subcore_barrier()

        def body(j, carry):
            kk = j * _NW + wid

            @pl.when(kk < nch)
            def _():
                off = kk * _C
                pltpu.sync_copy(alh.at[pl.ds(off, _C)], alb)
                pltpu.sync_copy(dsth.at[pl.ds(off, _C)], dstb)
                gv = gvb[...]
                for g in range(_C // 16):
                    ex16 = jnp.exp(alb[pl.ds(g * 16, 16)] - gv)
                    exb[pl.ds(g * 16, 16)] = ex16
                pltpu.sync_copy(exb, ex_out.at[pl.ds(off, _C)])
                pltpu.sync_copy(exb, acc.at[dstb], add=True)
            return carry

        lax.fori_loop(0, jmax, body, 0)
        plsc.subcore_barrier()
        pltpu.sync_copy(acc.at[pl.ds(sid * _SEG, _SEG)],
                        den_out.at[cid, pl.ds(sid * _SEG, _SEG)])

    return k


def _make_rowscat(E, scaled):
    nch = E // _C
    jmax = (nch + _NW - 1) // _NW

    scratch = [
        pltpu.VMEM((_C,), _i32),
        pltpu.VMEM((_C,), _i32),
        pltpu.VMEM((_C,), _f32),
        pltpu.VMEM((_C, _F), _f32),
        pltpu.VMEM_SHARED((_NP, _F), _f32),
        pltpu.SemaphoreType.DMA,
    ]
    if scaled:
        scratch.insert(3, pltpu.VMEM((_NP,), _f32))

    @functools.partial(
        pl.kernel,
        out_type=jax.ShapeDtypeStruct((_NC, _NP, _F), _f32),
        mesh=_sc_mesh(),
        compiler_params=_SC_PARAMS,
        scratch_types=scratch,
    )
    def k(table, srch, dsth, wh, *rest):
        if scaled:
            rdh, zm, out = rest[0], rest[1], rest[2]
            srcb, dstb, wb, rdv, rows, acc, sem = rest[3:]
        else:
            zm, out = rest[0], rest[1]
            srcb, dstb, wb, rows, acc, sem = rest[2:]
        cid = lax.axis_index("c")
        sid = lax.axis_index("s")
        wid = sid * _NC + cid
        pltpu.sync_copy(zm.at[pl.ds(sid * _SEG, _SEG)], acc.at[pl.ds(sid * _SEG, _SEG)])
        if scaled:
            pltpu.sync_copy(rdh, rdv)
        plsc.subcore_barrier()

        def body(j, carry):
            kk = j * _NW + wid

            @pl.when(kk < nch)
            def _():
                off = kk * _C
                pltpu.sync_copy(srch.at[pl.ds(off, _C)], srcb)
                pltpu.sync_copy(dsth.at[pl.ds(off, _C)], dstb)
                pltpu.sync_copy(wh.at[pl.ds(off, _C)], wb)
                pltpu.async_copy(table.at[srcb], rows, sem).wait()
                dn = lax.GatherDimensionNumbers(
                    offset_dims=(), collapsed_slice_dims=(0,),
                    start_index_map=(0,))
                for g in range(_C // 16):
                    w16 = wb[pl.ds(g * 16, 16)]
                    if scaled:
                        w16 = w16 * plsc.load_gather(rdv, [dstb[pl.ds(g * 16, 16)]])
                    for i in range(16):
                        e = g * 16 + i
                        wvi = lax.gather(
                            w16, jnp.full((16, 1), i, _i32), dn, (1,),
                            mode=lax.GatherScatterMode.PROMISE_IN_BOUNDS)
                        for c in range(_F // 16):
                            rows[e, pl.ds(c * 16, 16)] = rows[e, pl.ds(c * 16, 16)] * wvi
                pltpu.sync_copy(rows, acc.at[dstb], add=True)
            return carry

        lax.fori_loop(0, jmax, body, 0)
        plsc.subcore_barrier()
        pltpu.sync_copy(acc.at[pl.ds(sid * _SEG, _SEG)],
                        out.at[cid, pl.ds(sid * _SEG, _SEG)])

    return k


def _vspec():
    return pl.BlockSpec((_BM, 1), lambda m: (m, 0))


def _mspec():
    return pl.BlockSpec((_BM, _F), lambda m: (m, 0))


def _wspec():
    return pl.BlockSpec((_F, _F), lambda m: (0, 0))


def _bspec():
    return pl.BlockSpec((1, _F), lambda m: (0, 0))


def _sspec():
    return pl.BlockSpec((1, 1), lambda m: (0, 0))


def _accspec(i):
    return pl.BlockSpec((None, _BM, _F), lambda m, i=i: (i, m, 0))


def _dot(a, b):
    return jnp.dot(a, b, preferred_element_type=_f32)


def _t1_body(x_ref, w_ref, dinv_ref, y_ref):
    y_ref[...] = _dot(x_ref[...], w_ref[...]) * dinv_ref[...]


def _t1(x, W, dinv_col):
    return pl.pallas_call(
        _t1_body,
        grid=(_GRID,),
        in_specs=[_mspec(), _wspec(), _vspec()],
        out_specs=_mspec(),
        out_shape=jax.ShapeDtypeStruct((_N, _F), _f32),
    )(x, W, dinv_col)


def _t2_body(a0, a1, y1, dinv, b1, wg, asv, adv, cem,
             h1o, xwo, aso, ado, selo):
    h1 = jnp.maximum((a0[...] + a1[...] + y1[...]) * dinv[...] + b1[...], 0.0)
    xw = _dot(h1, wg[...])
    a_s = _dot(xw, asv[...])
    a_d = _dot(xw, adv[...])
    asl = a_s + a_d + cem[0, 0]
    asl = jnp.maximum(asl, 0.2 * asl)
    h1o[...] = h1
    xwo[...] = xw
    aso[...] = a_s
    ado[...] = a_d
    selo[...] = asl


def _t2(acc, y1, dinv_col, b1, Wg, att_src_col, att_dst_col, cem):
    return pl.pallas_call(
        _t2_body,
        grid=(_GRID,),
        in_specs=[_accspec(0), _accspec(1), _mspec(), _vspec(), _bspec(),
                  _wspec(),
                  pl.BlockSpec((_F, 1), lambda m: (0, 0)),
                  pl.BlockSpec((_F, 1), lambda m: (0, 0)),
                  _sspec()],
        out_specs=[_mspec(), _mspec(), _vspec(), _vspec(), _vspec()],
        out_shape=[
            jax.ShapeDtypeStruct((_N, _F), _f32),
            jax.ShapeDtypeStruct((_N, _F), _f32),
            jax.ShapeDtypeStruct((_N, 1), _f32),
            jax.ShapeDtypeStruct((_N, 1), _f32),
            jax.ShapeDtypeStruct((_N, 1), _f32),
        ],
    )(acc, acc, y1, dinv_col, b1, Wg, att_src_col, att_dst_col, cem)


def _t3_body(d0, d1, asl, g, rdo, swo):
    exs = jnp.exp(asl[...] - g[0, 0])
    rd = 1.0 / (d0[...] + d1[...] + exs + 1e-16)
    rdo[...] = rd
    swo[...] = exs * rd


def _t3(dacc_v, aself_v, g):
    blk = pl.BlockSpec((_NPR, 128), lambda i: (0, 0))
    return pl.pallas_call(
        _t3_body,
        grid=(1,),
        in_specs=[pl.BlockSpec((None, _NPR, 128), lambda i: (0, 0, 0)),
                  pl.BlockSpec((None, _NPR, 128), lambda i: (1, 0, 0)),
                  blk, pl.BlockSpec((1, 1), lambda i: (0, 0))],
        out_specs=[blk, blk],
        out_shape=[jax.ShapeDtypeStruct((_NPR, 128), _f32),
                   jax.ShapeDtypeStruct((_NPR, 128), _f32)],
    )(dacc_v, dacc_v, aself_v, g)


def _t4_body(g0, g1, xw2, selfw, bg, w3, dinv, y3o):
    ha = jnp.maximum(g0[...] + g1[...] + selfw[...] * xw2[...] + bg[...], 0.0)
    y3o[...] = _dot(ha, w3[...]) * dinv[...]


def _t4(gacc, xw2, selfw_col, bg, W3, dinv_col):
    return pl.pallas_call(
        _t4_body,
        grid=(_GRID,),
        in_specs=[_accspec(0), _accspec(1), _mspec(), _vspec(), _bspec(),
                  _wspec(), _vspec()],
        out_specs=_mspec(),
        out_shape=jax.ShapeDtypeStruct((_N, _F), _f32),
    )(gacc, gacc, xw2, selfw_col, bg, W3, dinv_col)


def _t5_body(a0, a1, y3, dinv, b3, h1, qwt, kwt, vwt, qb, kb, vb, cs, feao):
    h2 = jnp.maximum((a0[...] + a1[...] + y3[...]) * dinv[...] + b3[...], 0.0)
    h1v = h1[...]
    Q0 = _dot(h1v, qwt[...]) + qb[...]
    Q1 = _dot(h2, qwt[...]) + qb[...]
    K0 = _dot(h1v, kwt[...]) + kb[...]
    K1 = _dot(h2, kwt[...]) + kb[...]
    V0 = _dot(h1v, vwt[...]) + vb[...]
    V1 = _dot(h2, vwt[...]) + vb[...]
    sc = 1.0 / sqrt(float(_F))
    s00 = jnp.sum(Q0 * K0, axis=1, keepdims=True) * sc
    s01 = jnp.sum(Q0 * K1, axis=1, keepdims=True) * sc
    s10 = jnp.sum(Q1 * K0, axis=1, keepdims=True) * sc
    s11 = jnp.sum(Q1 * K1, axis=1, keepdims=True) * sc
    c0 = cs[0, 0]
    c1 = cs[0, 1]
    cb = cs[0, 2]
    m0 = jnp.maximum(s00, s10)
    e00 = jnp.exp(s00 - m0)
    e10 = jnp.exp(s10 - m0)
    a00 = e00 / (e00 + e10)
    a10 = e10 / (e00 + e10)
    m1 = jnp.maximum(s01, s11)
    e01 = jnp.exp(s01 - m1)
    e11 = jnp.exp(s11 - m1)
    a01 = e01 / (e01 + e11)
    a11 = e11 / (e01 + e11)
    g0 = c0 * a00 + c1 * a10
    g1 = c0 * a01 + c1 * a11
    feao[...] = g0 * V0 + g1 * V1 + cb


def _t5(acc2, y3, dinv_col, b3, h1, qWT, kWT, vWT, qb, kb, vb, cs):
    return pl.pallas_call(
        _t5_body,
        grid=(_GRID,),
        in_specs=[_accspec(0), _accspec(1), _mspec(), _vspec(), _bspec(),
                  _mspec(), _wspec(), _wspec(), _wspec(),
                  _bspec(), _bspec(), _bspec(),
                  pl.BlockSpec((1, 3), lambda m: (0, 0))],
        out_specs=_mspec(),
        out_shape=jax.ShapeDtypeStruct((_N, _F), _f32),
    )(acc2, acc2, y3, dinv_col, b3, h1, qWT, kWT, vWT, qb, kb, vb, cs)


def _t6_body(a_ref, b_ref, o_ref):
    o_ref[...] = lax.dot_general(a_ref[...], b_ref[...],
                                 (((1,), (1,)), ((), ())),
                                 preferred_element_type=_f32)


_BT = 400


def _t6(A, B):
    return pl.pallas_call(
        _t6_body,
        grid=(_N // _BT,),
        in_specs=[pl.BlockSpec((_BT, _F), lambda i: (i, 0)),
                  pl.BlockSpec((_N, _F), lambda i: (0, 0))],
        out_specs=pl.BlockSpec((_BT, _N), lambda i: (i, 0)),
        out_shape=jax.ShapeDtypeStruct((_N, _N), _f32),
    )(A, B)


def _tdinv_body(d0, d1, o_ref):
    o_ref[...] = lax.rsqrt(d0[...] + d1[...] + 1.0)


def _tdinv(deg_v):
    blk = pl.BlockSpec((_NPR, 128), lambda i: (0, 0))
    return pl.pallas_call(
        _tdinv_body,
        grid=(1,),
        in_specs=[pl.BlockSpec((None, _NPR, 128), lambda i: (0, 0, 0)),
                  pl.BlockSpec((None, _NPR, 128), lambda i: (1, 0, 0))],
        out_specs=blk,
        out_shape=jax.ShapeDtypeStruct((_NPR, 128), _f32),
    )(deg_v, deg_v)


def _flat_pad(col):
    return jnp.pad(col[:, 0], (0, _NP - _N))


def _branch(x, edges, dm, p1, pg, p2, zv, zm):
    E = edges.shape[1]
    src = edges[0].astype(_i32)
    dst = edges[1].astype(_i32)
    dm128 = jnp.reshape(dm, (_N * _N // 128, 128))

    ew, degp, sums = _make_ew_deg(E)(dm128, src, dst, zv)
    mean_ew = jnp.sum(sums) / E

    dinv_v = _tdinv(jnp.reshape(degp, (_NC, _NPR, 128)))
    dinv_flat = jnp.reshape(dinv_v, (_NP,))
    dinv_col = dinv_flat[:_N, None]

    y1 = _t1(x, p1["W"], dinv_col)
    acc = _make_rowscat(E, False)(y1, src, dst, ew, zm)
    ce = jnp.sum(pg["W_e"][0] * pg["att_edge"])
    h1, xw2, as_c, ad_c, asl_c = _t2(
        acc, y1, dinv_col, p1["b"][None, :], pg["W"],
        pg["att_src"][:, None], pg["att_dst"][:, None],
        jnp.reshape(ce * mean_ew, (1, 1)))
    asrc_flat = _flat_pad(as_c)
    adst_flat = _flat_pad(ad_c)
    cvec = jnp.broadcast_to(ce[None], (16,))

    alpha, amax = _make_alpha(E)(asrc_flat, adst_flat, ew, src, dst, cvec)
    g = jnp.maximum(jnp.max(amax), jnp.max(asl_c))
    gvec = jnp.broadcast_to(g[None], (16,))

    ex, denp = _make_denom(E)(alpha, dst, gvec, zv)
    rd_v, selfw_v = _t3(jnp.reshape(denp, (_NC, _NPR, 128)),
                        jnp.reshape(_flat_pad(asl_c), (_NPR, 128)),
                        jnp.reshape(g, (1, 1)))
    rd_flat = jnp.reshape(rd_v, (_NP,))
    selfw_col = jnp.reshape(selfw_v, (_NP,))[:_N, None]

    gacc = _make_rowscat(E, True)(xw2, src, dst, ex, rd_flat, zm)
    y3 = _t4(gacc, xw2, selfw_col, pg["b"][None, :], p2["W"], dinv_col)

    acc2 = _make_rowscat(E, False)(y3, src, dst, ew, zm)
    return h1, acc2, y3, dinv_col


def _att_branch(h1, acc2, y3, dinv_col, p2b, pa):
    cs = jnp.reshape(jnp.stack([pa["cW"][0, 0, 0], pa["cW"][0, 1, 0],
                                pa["cb"][0]]), (1, 3))
    return _t5(acc2, y3, dinv_col, p2b[None, :], h1,
               pa["qW"].T, pa["kW"].T, pa["vW"].T,
               pa["qb"][None, :], pa["kb"][None, :], pa["vb"][None, :], cs)


def kernel(ll_edges, ll_data_matrix, dd_edges, dd_data_matrix, params):
    kx = jax.random.key(1)
    x_lnc = jax.random.normal(jax.random.fold_in(kx, 0), (_N, _F), dtype=_f32)
    x_dis = jax.random.normal(jax.random.fold_in(kx, 1), (_N, _F), dtype=_f32)
    zv = jnp.zeros((_NP,), _f32)
    zm = jnp.zeros((_NP, _F), _f32)

    l_parts = _branch(x_lnc, ll_edges, ll_data_matrix,
                      params["gcn_lnc1"], params["gat_lnc1"],
                      params["gcn_lnc2"], zv, zm)
    d_parts = _branch(x_dis, dd_edges, dd_data_matrix,
                      params["gcn_dis1"], params["gat_dis1"],
                      params["gcn_dis2"], zv, zm)
    lnc = _att_branch(*l_parts, params["gcn_lnc2"]["b"], params["att_lnc"])
    dis = _att_branch(*d_parts, params["gcn_dis2"]["b"], params["att_dis"])
    return _t6(lnc, dis), lnc, dis


# speedup vs baseline: 10.0478x; 1.0422x over previous
import functools
from math import sqrt

import jax
import jax.numpy as jnp
from jax import lax
from jax.experimental import pallas as pl
from jax.experimental.pallas import tpu as pltpu
from jax.experimental.pallas import tpu_sc as plsc

_N = 10000
_F = 128
_NP = 10240
_C = 128
_NC = 2
_NS = 16
_NW = _NC * _NS
_BM = 1000
_GRID = _N // _BM
_SEG = _NP // _NS
_NPR = _NP // 128

_f32 = jnp.float32
_i32 = jnp.int32


def _sc_mesh():
    return plsc.VectorSubcoreMesh(core_axis_name="c", subcore_axis_name="s", num_cores=_NC, num_subcores=_NS)


_SC_PARAMS = pltpu.CompilerParams(needs_layout_passes=False)


def _make_ew_deg(E, E_real):
    nch = E // _C
    nch_real = E_real // _C
    jmax = nch // _NW

    @functools.partial(
        pl.kernel,
        out_type=(
            jax.ShapeDtypeStruct((E,), _f32),
            jax.ShapeDtypeStruct((_NC, _NP), _f32),
            jax.ShapeDtypeStruct((_NW, 16), _f32),
        ),
        mesh=_sc_mesh(),
        compiler_params=_SC_PARAMS,
        scratch_types=[
            pltpu.VMEM((_C,), _i32),
            pltpu.VMEM((_C,), _i32),
            pltpu.VMEM((_C,), _i32),
            pltpu.VMEM((_C,), _i32),
            pltpu.VMEM((_C, 128), _f32),
            pltpu.VMEM((_C,), _f32),
            pltpu.VMEM((16,), _f32),
            pltpu.VMEM_SHARED((_NP,), _f32),
            pltpu.SemaphoreType.DMA,
        ],
    )
    def k(dm128, srch, dsth, zv, ew_out, deg_out, sum_out,
          srcb, dstb, rowb, laneb, dmrows, ewb, sumb, acc, sem):
        cid = lax.axis_index("c")
        sid = lax.axis_index("s")
        wid = sid * _NC + cid
        pltpu.sync_copy(zv.at[pl.ds(sid * _SEG, _SEG)], acc.at[pl.ds(sid * _SEG, _SEG)])
        sumb[...] = jnp.zeros((16,), _f32)
        plsc.subcore_barrier()

        def body(j, carry):
            kk = j * _NW + wid
            off = kk * _C
            pltpu.sync_copy(srch.at[pl.ds(off, _C)], srcb)
            pltpu.sync_copy(dsth.at[pl.ds(off, _C)], dstb)
            for g in range(_C // 16):
                s16 = srcb[pl.ds(g * 16, 16)]
                d16 = jnp.minimum(dstb[pl.ds(g * 16, 16)], _N - 1)
                flat = s16 * _N + d16
                rowb[pl.ds(g * 16, 16)] = lax.shift_right_logical(flat, 7)
                laneb[pl.ds(g * 16, 16)] = lax.bitwise_and(flat, 127)
            pltpu.async_copy(dm128.at[rowb], dmrows, sem).wait()
            for g in range(_C // 16):
                rloc = lax.iota(_i32, 16) + g * 16
                ew16 = plsc.load_gather(dmrows, [rloc, laneb[pl.ds(g * 16, 16)]])
                ewb[pl.ds(g * 16, 16)] = ew16

            @pl.when(kk < nch_real)
            def _():
                sacc = jnp.zeros((16,), _f32)
                for g in range(_C // 16):
                    sacc = sacc + ewb[pl.ds(g * 16, 16)]
                sumb[...] = sumb[...] + sacc
            pltpu.sync_copy(ewb, ew_out.at[pl.ds(off, _C)])
            pltpu.sync_copy(ewb, acc.at[dstb], add=True)
            return carry

        lax.fori_loop(0, jmax, body, 0)
        pltpu.sync_copy(sumb, sum_out.at[wid])
        plsc.subcore_barrier()
        pltpu.sync_copy(acc.at[pl.ds(sid * _SEG, _SEG)],
                        deg_out.at[cid, pl.ds(sid * _SEG, _SEG)])

    return k


def _make_alpha(E):
    nch = E // _C
    jmax = nch // _NW

    @functools.partial(
        pl.kernel,
        out_type=(
            jax.ShapeDtypeStruct((E,), _f32),
            jax.ShapeDtypeStruct((_NW, 16), _f32),
        ),
        mesh=_sc_mesh(),
        compiler_params=_SC_PARAMS,
        scratch_types=[
            pltpu.VMEM((_NP,), _f32),
            pltpu.VMEM((_NP,), _f32),
            pltpu.VMEM((_C,), _i32),
            pltpu.VMEM((_C,), _i32),
            pltpu.VMEM((_C,), _f32),
            pltpu.VMEM((_C,), _f32),
            pltpu.VMEM((16,), _f32),
            pltpu.VMEM((16,), _f32),
        ],
    )
    def k(asrch, adsth, ewh, srch, dsth, cvh, al_out, max_out,
          asv, adv, srcb, dstb, ewb, alb, maxb, cvb):
        cid = lax.axis_index("c")
        sid = lax.axis_index("s")
        wid = sid * _NC + cid
        pltpu.sync_copy(asrch, asv)
        pltpu.sync_copy(adsth, adv)
        pltpu.sync_copy(cvh, cvb)
        maxb[...] = jnp.full((16,), -3.0e38, _f32)

        def body(j, carry):
            kk = j * _NW + wid
            off = kk * _C
            pltpu.sync_copy(srch.at[pl.ds(off, _C)], srcb)
            pltpu.sync_copy(dsth.at[pl.ds(off, _C)], dstb)
            pltpu.sync_copy(ewh.at[pl.ds(off, _C)], ewb)
            cv = cvb[...]
            for g in range(_C // 16):
                s16 = srcb[pl.ds(g * 16, 16)]
                d16 = dstb[pl.ds(g * 16, 16)]
                ew16 = ewb[pl.ds(g * 16, 16)]
                a16 = (plsc.load_gather(asv, [s16])
                       + plsc.load_gather(adv, [d16]) + cv * ew16)
                a16 = jnp.maximum(a16, 0.2 * a16)
                alb[pl.ds(g * 16, 16)] = a16
                maxb[...] = jnp.maximum(maxb[...], a16)
            pltpu.sync_copy(alb, al_out.at[pl.ds(off, _C)])
            return carry

        lax.fori_loop(0, jmax, body, 0)
        pltpu.sync_copy(maxb, max_out.at[wid])

    return k


def _make_denom(E):
    nch = E // _C
    jmax = nch // _NW

    @functools.partial(
        pl.kernel,
        out_type=(
            jax.ShapeDtypeStruct((E,), _f32),
            jax.ShapeDtypeStruct((_NC, _NP), _f32),
        ),
        mesh=_sc_mesh(),
        compiler_params=_SC_PARAMS,
        scratch_types=[
            pltpu.VMEM((_C,), _f32),
            pltpu.VMEM((_C,), _f32),
            pltpu.VMEM((_C,), _i32),
            pltpu.VMEM((16,), _f32),
            pltpu.VMEM_SHARED((_NP,), _f32),
        ],
    )
    def k(alh, dsth, gvh, zv, ex_out, den_out, alb, exb, dstb, gvb, acc):
        cid = lax.axis_index("c")
        sid = lax.axis_index("s")
        wid = sid * _NC + cid
        pltpu.sync_copy(zv.at[pl.ds(sid * _SEG, _SEG)], acc.at[pl.ds(sid * _SEG, _SEG)])
        pltpu.sync_copy(gvh, gvb)
        plsc.subcore_barrier()

        def body(j, carry):
            kk = j * _NW + wid
            off = kk * _C
            pltpu.sync_copy(alh.at[pl.ds(off, _C)], alb)
            pltpu.sync_copy(dsth.at[pl.ds(off, _C)], dstb)
            gv = gvb[...]
            for g in range(_C // 16):
                ex16 = jnp.exp(alb[pl.ds(g * 16, 16)] - gv)
                exb[pl.ds(g * 16, 16)] = ex16
            pltpu.sync_copy(exb, ex_out.at[pl.ds(off, _C)])
            pltpu.sync_copy(exb, acc.at[dstb], add=True)
            return carry

        lax.fori_loop(0, jmax, body, 0)
        plsc.subcore_barrier()
        pltpu.sync_copy(acc.at[pl.ds(sid * _SEG, _SEG)],
                        den_out.at[cid, pl.ds(sid * _SEG, _SEG)])

    return k


def _make_rowscat(E, scaled):
    nch = E // _C
    T = nch // _NW
    assert T * _NW == nch and T >= 4 and T % 2 == 0

    scratch = []
    for _ in range(2):
        scratch += [
            pltpu.VMEM((_C,), _i32),
            pltpu.VMEM((_C,), _i32),
            pltpu.VMEM((_C,), _f32),
            pltpu.VMEM((_C, _F), _f32),
            pltpu.SemaphoreType.DMA,
        ]
    if scaled:
        scratch.append(pltpu.VMEM((_NP,), _f32))
    scratch.append(pltpu.VMEM_SHARED((_NP, _F), _f32))

    @functools.partial(
        pl.kernel,
        out_type=jax.ShapeDtypeStruct((_NC, _NP, _F), _f32),
        mesh=_sc_mesh(),
        compiler_params=_SC_PARAMS,
        scratch_types=scratch,
    )
    def k(table, srch, dsth, wh, *rest):
        if scaled:
            rdh, zm, out = rest[0], rest[1], rest[2]
            rest = rest[3:]
        else:
            zm, out = rest[0], rest[1]
            rest = rest[2:]
        bufs = [rest[i * 5:(i + 1) * 5] for i in range(2)]
        if scaled:
            rdv = rest[10]
            acc = rest[11]
        else:
            acc = rest[10]
        cid = lax.axis_index("c")
        sid = lax.axis_index("s")
        wid = sid * _NC + cid
        pltpu.sync_copy(zm.at[pl.ds(sid * _SEG, _SEG)], acc.at[pl.ds(sid * _SEG, _SEG)])
        if scaled:
            pltpu.sync_copy(rdh, rdv)
        plsc.subcore_barrier()

        dn = lax.GatherDimensionNumbers(
            offset_dims=(), collapsed_slice_dims=(0,), start_index_map=(0,))

        def load_idx(t, b):
            off = (t * _NW + wid) * _C
            pltpu.sync_copy(srch.at[pl.ds(off, _C)], b[0])
            pltpu.sync_copy(dsth.at[pl.ds(off, _C)], b[1])
            pltpu.sync_copy(wh.at[pl.ds(off, _C)], b[2])

        def gstart(b):
            pltpu.async_copy(table.at[b[0]], b[3], b[4])

        def gwait(b):
            pltpu.make_async_copy(table.at[b[0]], b[3], b[4]).wait()

        def scale(b):
            rows = b[3]
            for g in range(_C // 16):
                w16 = b[2][pl.ds(g * 16, 16)]
                if scaled:
                    w16 = w16 * plsc.load_gather(rdv, [b[1][pl.ds(g * 16, 16)]])
                for i in range(16):
                    e = g * 16 + i
                    wvi = lax.gather(
                        w16, jnp.full((16, 1), i, _i32), dn, (1,),
                        mode=lax.GatherScatterMode.PROMISE_IN_BOUNDS)
                    for c in range(_F // 16):
                        rows[e, pl.ds(c * 16, 16)] = rows[e, pl.ds(c * 16, 16)] * wvi

        def step(t, cur, last):
            gwait(cur)
            scale(cur)
            pltpu.sync_copy(cur[3], acc.at[cur[1]], add=True)
            if not last:
                load_idx(t + 2, cur)
                gstart(cur)

        load_idx(0, bufs[0])
        gstart(bufs[0])
        load_idx(1, bufs[1])
        gstart(bufs[1])

        def body(u, carry):
            t = u * 2
            step(t, bufs[0], False)
            step(t + 1, bufs[1], False)
            return carry

        lax.fori_loop(0, (T - 2) // 2, body, 0)
        step(T - 2, bufs[0], True)
        step(T - 1, bufs[1], True)
        plsc.subcore_barrier()
        pltpu.sync_copy(acc.at[pl.ds(sid * _SEG, _SEG)],
                        out.at[cid, pl.ds(sid * _SEG, _SEG)])

    return k


def _vspec():
    return pl.BlockSpec((_BM, 1), lambda m: (m, 0))


def _mspec():
    return pl.BlockSpec((_BM, _F), lambda m: (m, 0))


def _wspec():
    return pl.BlockSpec((_F, _F), lambda m: (0, 0))


def _bspec():
    return pl.BlockSpec((1, _F), lambda m: (0, 0))


def _sspec():
    return pl.BlockSpec((1, 1), lambda m: (0, 0))


def _accspec(i):
    return pl.BlockSpec((None, _BM, _F), lambda m, i=i: (i, m, 0))


def _dot(a, b):
    return jnp.dot(a, b, preferred_element_type=_f32)


def _t1_body(x_ref, w_ref, dinv_ref, y_ref):
    y_ref[...] = _dot(x_ref[...], w_ref[...]) * dinv_ref[...]


def _t1(x, W, dinv_col):
    return pl.pallas_call(
        _t1_body,
        grid=(_GRID,),
        in_specs=[_mspec(), _wspec(), _vspec()],
        out_specs=_mspec(),
        out_shape=jax.ShapeDtypeStruct((_N, _F), _f32),
    )(x, W, dinv_col)


def _t2_body(a0, a1, y1, dinv, b1, wg, asv, adv, cem,
             h1o, xwo, aso, ado, selo):
    h1 = jnp.maximum((a0[...] + a1[...] + y1[...]) * dinv[...] + b1[...], 0.0)
    xw = _dot(h1, wg[...])
    a_s = _dot(xw, asv[...])
    a_d = _dot(xw, adv[...])
    asl = a_s + a_d + cem[0, 0]
    asl = jnp.maximum(asl, 0.2 * asl)
    h1o[...] = h1
    xwo[...] = xw
    aso[...] = a_s
    ado[...] = a_d
    selo[...] = asl


def _t2(acc, y1, dinv_col, b1, Wg, att_src_col, att_dst_col, cem):
    return pl.pallas_call(
        _t2_body,
        grid=(_GRID,),
        in_specs=[_accspec(0), _accspec(1), _mspec(), _vspec(), _bspec(),
                  _wspec(),
                  pl.BlockSpec((_F, 1), lambda m: (0, 0)),
                  pl.BlockSpec((_F, 1), lambda m: (0, 0)),
                  _sspec()],
        out_specs=[_mspec(), _mspec(), _vspec(), _vspec(), _vspec()],
        out_shape=[
            jax.ShapeDtypeStruct((_N, _F), _f32),
            jax.ShapeDtypeStruct((_N, _F), _f32),
            jax.ShapeDtypeStruct((_N, 1), _f32),
            jax.ShapeDtypeStruct((_N, 1), _f32),
            jax.ShapeDtypeStruct((_N, 1), _f32),
        ],
    )(acc, acc, y1, dinv_col, b1, Wg, att_src_col, att_dst_col, cem)


def _t3_body(d0, d1, asl, g, rdo, swo):
    exs = jnp.exp(asl[...] - g[0, 0])
    rd = 1.0 / (d0[...] + d1[...] + exs + 1e-16)
    rdo[...] = rd
    swo[...] = exs * rd


def _t3(dacc_v, aself_v, g):
    blk = pl.BlockSpec((_NPR, 128), lambda i: (0, 0))
    return pl.pallas_call(
        _t3_body,
        grid=(1,),
        in_specs=[pl.BlockSpec((None, _NPR, 128), lambda i: (0, 0, 0)),
                  pl.BlockSpec((None, _NPR, 128), lambda i: (1, 0, 0)),
                  blk, pl.BlockSpec((1, 1), lambda i: (0, 0))],
        out_specs=[blk, blk],
        out_shape=[jax.ShapeDtypeStruct((_NPR, 128), _f32),
                   jax.ShapeDtypeStruct((_NPR, 128), _f32)],
    )(dacc_v, dacc_v, aself_v, g)


def _t4_body(g0, g1, xw2, selfw, bg, w3, dinv, y3o):
    ha = jnp.maximum(g0[...] + g1[...] + selfw[...] * xw2[...] + bg[...], 0.0)
    y3o[...] = _dot(ha, w3[...]) * dinv[...]


def _t4(gacc, xw2, selfw_col, bg, W3, dinv_col):
    return pl.pallas_call(
        _t4_body,
        grid=(_GRID,),
        in_specs=[_accspec(0), _accspec(1), _mspec(), _vspec(), _bspec(),
                  _wspec(), _vspec()],
        out_specs=_mspec(),
        out_shape=jax.ShapeDtypeStruct((_N, _F), _f32),
    )(gacc, gacc, xw2, selfw_col, bg, W3, dinv_col)


def _t5_body(a0, a1, y3, dinv, b3, h1, qwt, kwt, vwt, qb, kb, vb, cs, feao):
    h2 = jnp.maximum((a0[...] + a1[...] + y3[...]) * dinv[...] + b3[...], 0.0)
    h1v = h1[...]
    Q0 = _dot(h1v, qwt[...]) + qb[...]
    Q1 = _dot(h2, qwt[...]) + qb[...]
    K0 = _dot(h1v, kwt[...]) + kb[...]
    K1 = _dot(h2, kwt[...]) + kb[...]
    V0 = _dot(h1v, vwt[...]) + vb[...]
    V1 = _dot(h2, vwt[...]) + vb[...]
    sc = 1.0 / sqrt(float(_F))
    s00 = jnp.sum(Q0 * K0, axis=1, keepdims=True) * sc
    s01 = jnp.sum(Q0 * K1, axis=1, keepdims=True) * sc
    s10 = jnp.sum(Q1 * K0, axis=1, keepdims=True) * sc
    s11 = jnp.sum(Q1 * K1, axis=1, keepdims=True) * sc
    c0 = cs[0, 0]
    c1 = cs[0, 1]
    cb = cs[0, 2]
    m0 = jnp.maximum(s00, s10)
    e00 = jnp.exp(s00 - m0)
    e10 = jnp.exp(s10 - m0)
    a00 = e00 / (e00 + e10)
    a10 = e10 / (e00 + e10)
    m1 = jnp.maximum(s01, s11)
    e01 = jnp.exp(s01 - m1)
    e11 = jnp.exp(s11 - m1)
    a01 = e01 / (e01 + e11)
    a11 = e11 / (e01 + e11)
    g0 = c0 * a00 + c1 * a10
    g1 = c0 * a01 + c1 * a11
    feao[...] = g0 * V0 + g1 * V1 + cb


def _t5(acc2, y3, dinv_col, b3, h1, qWT, kWT, vWT, qb, kb, vb, cs):
    return pl.pallas_call(
        _t5_body,
        grid=(_GRID,),
        in_specs=[_accspec(0), _accspec(1), _mspec(), _vspec(), _bspec(),
                  _mspec(), _wspec(), _wspec(), _wspec(),
                  _bspec(), _bspec(), _bspec(),
                  pl.BlockSpec((1, 3), lambda m: (0, 0))],
        out_specs=_mspec(),
        out_shape=jax.ShapeDtypeStruct((_N, _F), _f32),
    )(acc2, acc2, y3, dinv_col, b3, h1, qWT, kWT, vWT, qb, kb, vb, cs)


def _t6_body(a_ref, b_ref, o_ref):
    o_ref[...] = lax.dot_general(a_ref[...], b_ref[...],
                                 (((1,), (1,)), ((), ())),
                                 preferred_element_type=_f32)


_BT = 400


def _t6(A, B):
    return pl.pallas_call(
        _t6_body,
        grid=(_N // _BT,),
        in_specs=[pl.BlockSpec((_BT, _F), lambda i: (i, 0)),
                  pl.BlockSpec((_N, _F), lambda i: (0, 0))],
        out_specs=pl.BlockSpec((_BT, _N), lambda i: (i, 0)),
        out_shape=jax.ShapeDtypeStruct((_N, _N), _f32),
    )(A, B)


def _tdinv_body(d0, d1, o_ref):
    o_ref[...] = lax.rsqrt(d0[...] + d1[...] + 1.0)


def _tdinv(deg_v):
    blk = pl.BlockSpec((_NPR, 128), lambda i: (0, 0))
    return pl.pallas_call(
        _tdinv_body,
        grid=(1,),
        in_specs=[pl.BlockSpec((None, _NPR, 128), lambda i: (0, 0, 0)),
                  pl.BlockSpec((None, _NPR, 128), lambda i: (1, 0, 0))],
        out_specs=blk,
        out_shape=jax.ShapeDtypeStruct((_NPR, 128), _f32),
    )(deg_v, deg_v)


def _flat_pad(col):
    return jnp.pad(col[:, 0], (0, _NP - _N))


def _branch(x, edges, dm, p1, pg, p2, zv, zm):
    E = edges.shape[1]
    Ep = -(-E // (2 * _C * _NW)) * (2 * _C * _NW)
    npad = Ep - E
    pad_src = (jnp.arange(npad, dtype=_i32) * 97) % _N
    pad_dst = _N + (jnp.arange(npad, dtype=_i32) % (_NP - _N))
    src = jnp.concatenate([edges[0].astype(_i32), pad_src])
    dst = jnp.concatenate([edges[1].astype(_i32), pad_dst])
    dm128 = jnp.reshape(dm, (_N * _N // 128, 128))

    ew, degp, sums = _make_ew_deg(Ep, E)(dm128, src, dst, zv)
    mean_ew = jnp.sum(sums) / E

    dinv_v = _tdinv(jnp.reshape(degp, (_NC, _NPR, 128)))
    dinv_flat = jnp.reshape(dinv_v, (_NP,))
    dinv_col = dinv_flat[:_N, None]

    y1 = _t1(x, p1["W"], dinv_col)
    acc = _make_rowscat(Ep, False)(y1, src, dst, ew, zm)
    ce = jnp.sum(pg["W_e"][0] * pg["att_edge"])
    h1, xw2, as_c, ad_c, asl_c = _t2(
        acc, y1, dinv_col, p1["b"][None, :], pg["W"],
        pg["att_src"][:, None], pg["att_dst"][:, None],
        jnp.reshape(ce * mean_ew, (1, 1)))
    asrc_flat = _flat_pad(as_c)
    adst_flat = _flat_pad(ad_c)
    cvec = jnp.broadcast_to(ce[None], (16,))

    alpha, amax = _make_alpha(Ep)(asrc_flat, adst_flat, ew, src, dst, cvec)
    g = jnp.maximum(jnp.max(amax), jnp.max(asl_c))
    gvec = jnp.broadcast_to(g[None], (16,))

    ex, denp = _make_denom(Ep)(alpha, dst, gvec, zv)
    rd_v, selfw_v = _t3(jnp.reshape(denp, (_NC, _NPR, 128)),
                        jnp.reshape(_flat_pad(asl_c), (_NPR, 128)),
                        jnp.reshape(g, (1, 1)))
    rd_flat = jnp.reshape(rd_v, (_NP,))
    selfw_col = jnp.reshape(selfw_v, (_NP,))[:_N, None]

    gacc = _make_rowscat(Ep, True)(xw2, src, dst, ex, rd_flat, zm)
    y3 = _t4(gacc, xw2, selfw_col, pg["b"][None, :], p2["W"], dinv_col)

    acc2 = _make_rowscat(Ep, False)(y3, src, dst, ew, zm)
    return h1, acc2, y3, dinv_col


def _att_branch(h1, acc2, y3, dinv_col, p2b, pa):
    cs = jnp.reshape(jnp.stack([pa["cW"][0, 0, 0], pa["cW"][0, 1, 0],
                                pa["cb"][0]]), (1, 3))
    return _t5(acc2, y3, dinv_col, p2b[None, :], h1,
               pa["qW"].T, pa["kW"].T, pa["vW"].T,
               pa["qb"][None, :], pa["kb"][None, :], pa["vb"][None, :], cs)


def kernel(ll_edges, ll_data_matrix, dd_edges, dd_data_matrix, params):
    kx = jax.random.key(1)
    x_lnc = jax.random.normal(jax.random.fold_in(kx, 0), (_N, _F), dtype=_f32)
    x_dis = jax.random.normal(jax.random.fold_in(kx, 1), (_N, _F), dtype=_f32)
    zv = jnp.zeros((_NP,), _f32)
    zm = jnp.zeros((_NP, _F), _f32)

    l_parts = _branch(x_lnc, ll_edges, ll_data_matrix,
                      params["gcn_lnc1"], params["gat_lnc1"],
                      params["gcn_lnc2"], zv, zm)
    d_parts = _branch(x_dis, dd_edges, dd_data_matrix,
                      params["gcn_dis1"], params["gat_dis1"],
                      params["gcn_dis2"], zv, zm)
    lnc = _att_branch(*l_parts, params["gcn_lnc2"]["b"], params["att_lnc"])
    dis = _att_branch(*d_parts, params["gcn_dis2"]["b"], params["att_dis"])
    return _t6(lnc, dis), lnc, dis


# speedup vs baseline: 10.5271x; 1.0477x over previous
import functools
from math import sqrt

import jax
import jax.numpy as jnp
from jax import lax
from jax.experimental import pallas as pl
from jax.experimental.pallas import tpu as pltpu
from jax.experimental.pallas import tpu_sc as plsc

_N = 10000
_F = 128
_NP = 10240
_C = 128
_NC = 2
_NS = 16
_NW = _NC * _NS
_BM = 1000
_GRID = _N // _BM
_SEG = _NP // _NS
_NPM = 10112
_SEGM = _NPM // _NS
_NPR = _NP // 128

_f32 = jnp.float32
_i32 = jnp.int32


def _sc_mesh():
    return plsc.VectorSubcoreMesh(core_axis_name="c", subcore_axis_name="s", num_cores=_NC, num_subcores=_NS)


_SC_PARAMS = pltpu.CompilerParams(needs_layout_passes=False)


def _make_ew_deg(E, E_real):
    nch = E // _C
    nch_real = E_real // _C
    jmax = nch // _NW

    @functools.partial(
        pl.kernel,
        out_type=(
            jax.ShapeDtypeStruct((E,), _f32),
            jax.ShapeDtypeStruct((_NC, _NP), _f32),
            jax.ShapeDtypeStruct((_NW, 16), _f32),
        ),
        mesh=_sc_mesh(),
        compiler_params=_SC_PARAMS,
        scratch_types=[
            pltpu.VMEM((_C,), _i32),
            pltpu.VMEM((_C,), _i32),
            pltpu.VMEM((_C,), _i32),
            pltpu.VMEM((_C,), _i32),
            pltpu.VMEM((_C, 128), _f32),
            pltpu.VMEM((_C,), _f32),
            pltpu.VMEM((16,), _f32),
            pltpu.VMEM_SHARED((_NP,), _f32),
            pltpu.SemaphoreType.DMA,
        ],
    )
    def k(dm128, srch, dsth, zv, ew_out, deg_out, sum_out,
          srcb, dstb, rowb, laneb, dmrows, ewb, sumb, acc, sem):
        cid = lax.axis_index("c")
        sid = lax.axis_index("s")
        wid = sid * _NC + cid
        pltpu.sync_copy(zv.at[pl.ds(sid * _SEG, _SEG)], acc.at[pl.ds(sid * _SEG, _SEG)])
        sumb[...] = jnp.zeros((16,), _f32)
        plsc.subcore_barrier()

        def body(j, carry):
            kk = j * _NW + wid
            off = kk * _C
            pltpu.sync_copy(srch.at[pl.ds(off, _C)], srcb)
            pltpu.sync_copy(dsth.at[pl.ds(off, _C)], dstb)
            for g in range(_C // 16):
                s16 = srcb[pl.ds(g * 16, 16)]
                d16 = jnp.minimum(dstb[pl.ds(g * 16, 16)], _N - 1)
                flat = s16 * _N + d16
                rowb[pl.ds(g * 16, 16)] = lax.shift_right_logical(flat, 7)
                laneb[pl.ds(g * 16, 16)] = lax.bitwise_and(flat, 127)
            pltpu.async_copy(dm128.at[rowb], dmrows, sem).wait()
            for g in range(_C // 16):
                rloc = lax.iota(_i32, 16) + g * 16
                ew16 = plsc.load_gather(dmrows, [rloc, laneb[pl.ds(g * 16, 16)]])
                ewb[pl.ds(g * 16, 16)] = ew16

            @pl.when(kk < nch_real)
            def _():
                sacc = jnp.zeros((16,), _f32)
                for g in range(_C // 16):
                    sacc = sacc + ewb[pl.ds(g * 16, 16)]
                sumb[...] = sumb[...] + sacc
            pltpu.sync_copy(ewb, ew_out.at[pl.ds(off, _C)])
            pltpu.sync_copy(ewb, acc.at[dstb], add=True)
            return carry

        lax.fori_loop(0, jmax, body, 0)
        pltpu.sync_copy(sumb, sum_out.at[wid])
        plsc.subcore_barrier()
        pltpu.sync_copy(acc.at[pl.ds(sid * _SEG, _SEG)],
                        deg_out.at[cid, pl.ds(sid * _SEG, _SEG)])

    return k


def _make_alpha(E):
    nch = E // _C
    jmax = nch // _NW

    @functools.partial(
        pl.kernel,
        out_type=(
            jax.ShapeDtypeStruct((E,), _f32),
            jax.ShapeDtypeStruct((_NW, 16), _f32),
        ),
        mesh=_sc_mesh(),
        compiler_params=_SC_PARAMS,
        scratch_types=[
            pltpu.VMEM((_NP,), _f32),
            pltpu.VMEM((_NP,), _f32),
            pltpu.VMEM((_C,), _i32),
            pltpu.VMEM((_C,), _i32),
            pltpu.VMEM((_C,), _f32),
            pltpu.VMEM((_C,), _f32),
            pltpu.VMEM((16,), _f32),
            pltpu.VMEM((16,), _f32),
        ],
    )
    def k(asrch, adsth, ewh, srch, dsth, cvh, al_out, max_out,
          asv, adv, srcb, dstb, ewb, alb, maxb, cvb):
        cid = lax.axis_index("c")
        sid = lax.axis_index("s")
        wid = sid * _NC + cid
        pltpu.sync_copy(asrch, asv)
        pltpu.sync_copy(adsth, adv)
        pltpu.sync_copy(cvh, cvb)
        maxb[...] = jnp.full((16,), -3.0e38, _f32)

        def body(j, carry):
            kk = j * _NW + wid
            off = kk * _C
            pltpu.sync_copy(srch.at[pl.ds(off, _C)], srcb)
            pltpu.sync_copy(dsth.at[pl.ds(off, _C)], dstb)
            pltpu.sync_copy(ewh.at[pl.ds(off, _C)], ewb)
            cv = cvb[...]
            for g in range(_C // 16):
                s16 = srcb[pl.ds(g * 16, 16)]
                d16 = dstb[pl.ds(g * 16, 16)]
                ew16 = ewb[pl.ds(g * 16, 16)]
                a16 = (plsc.load_gather(asv, [s16])
                       + plsc.load_gather(adv, [d16]) + cv * ew16)
                a16 = jnp.maximum(a16, 0.2 * a16)
                alb[pl.ds(g * 16, 16)] = a16
                maxb[...] = jnp.maximum(maxb[...], a16)
            pltpu.sync_copy(alb, al_out.at[pl.ds(off, _C)])
            return carry

        lax.fori_loop(0, jmax, body, 0)
        pltpu.sync_copy(maxb, max_out.at[wid])

    return k


def _make_denom(E):
    nch = E // _C
    jmax = nch // _NW

    @functools.partial(
        pl.kernel,
        out_type=(
            jax.ShapeDtypeStruct((E,), _f32),
            jax.ShapeDtypeStruct((_NC, _NP), _f32),
        ),
        mesh=_sc_mesh(),
        compiler_params=_SC_PARAMS,
        scratch_types=[
            pltpu.VMEM((_C,), _f32),
            pltpu.VMEM((_C,), _f32),
            pltpu.VMEM((_C,), _i32),
            pltpu.VMEM((16,), _f32),
            pltpu.VMEM_SHARED((_NP,), _f32),
        ],
    )
    def k(alh, dsth, gvh, zv, ex_out, den_out, alb, exb, dstb, gvb, acc):
        cid = lax.axis_index("c")
        sid = lax.axis_index("s")
        wid = sid * _NC + cid
        pltpu.sync_copy(zv.at[pl.ds(sid * _SEG, _SEG)], acc.at[pl.ds(sid * _SEG, _SEG)])
        pltpu.sync_copy(gvh, gvb)
        plsc.subcore_barrier()

        def body(j, carry):
            kk = j * _NW + wid
            off = kk * _C
            pltpu.sync_copy(alh.at[pl.ds(off, _C)], alb)
            pltpu.sync_copy(dsth.at[pl.ds(off, _C)], dstb)
            gv = gvb[...]
            for g in range(_C // 16):
                ex16 = jnp.exp(alb[pl.ds(g * 16, 16)] - gv)
                exb[pl.ds(g * 16, 16)] = ex16
            pltpu.sync_copy(exb, ex_out.at[pl.ds(off, _C)])
            pltpu.sync_copy(exb, acc.at[dstb], add=True)
            return carry

        lax.fori_loop(0, jmax, body, 0)
        plsc.subcore_barrier()
        pltpu.sync_copy(acc.at[pl.ds(sid * _SEG, _SEG)],
                        den_out.at[cid, pl.ds(sid * _SEG, _SEG)])

    return k


def _make_rowscat(Ep, E_real, scaled):
    CR = 96
    Er = -(-E_real // (2 * CR * _NW)) * (2 * CR * _NW)
    assert Er <= Ep
    nch = Er // CR
    T = nch // _NW
    assert T * _NW == nch and T >= 4 and T % 2 == 0

    scratch = []
    for _ in range(2):
        scratch += [
            pltpu.VMEM((CR,), _i32),
            pltpu.VMEM((CR,), _i32),
            pltpu.VMEM((CR,), _f32),
            pltpu.VMEM((CR, _F), _f32),
            pltpu.SemaphoreType.DMA,
        ]
    scratch += [
        pltpu.VMEM((CR, _F), _f32),
        pltpu.VMEM((CR,), _i32),
        pltpu.SemaphoreType.DMA,
    ]
    if scaled:
        scratch.append(pltpu.VMEM((_NP,), _f32))
    scratch.append(pltpu.VMEM_SHARED((_NPM, _F), _f32))

    @functools.partial(
        pl.kernel,
        out_type=jax.ShapeDtypeStruct((_NC, _NPM, _F), _f32),
        mesh=_sc_mesh(),
        compiler_params=_SC_PARAMS,
        scratch_types=scratch,
    )
    def k(table, srch, dsth, wh, *rest):
        if scaled:
            rdh, zm, out = rest[0], rest[1], rest[2]
            rest = rest[3:]
        else:
            zm, out = rest[0], rest[1]
            rest = rest[2:]
        bufs = [rest[i * 5:(i + 1) * 5] for i in range(2)]
        sbuf, dstS, ssem = rest[10], rest[11], rest[12]
        if scaled:
            rdv = rest[13]
            acc = rest[14]
        else:
            acc = rest[13]
        cid = lax.axis_index("c")
        sid = lax.axis_index("s")
        wid = sid * _NC + cid
        pltpu.sync_copy(zm.at[pl.ds(sid * _SEGM, _SEGM)], acc.at[pl.ds(sid * _SEGM, _SEGM)])
        if scaled:
            pltpu.sync_copy(rdh, rdv)
        plsc.subcore_barrier()

        dn = lax.GatherDimensionNumbers(
            offset_dims=(), collapsed_slice_dims=(0,), start_index_map=(0,))

        def load_idx(t, b):
            off = (t * _NW + wid) * CR
            pltpu.sync_copy(srch.at[pl.ds(off, CR)], b[0])
            pltpu.sync_copy(dsth.at[pl.ds(off, CR)], b[1])
            pltpu.sync_copy(wh.at[pl.ds(off, CR)], b[2])

        def gstart(b):
            pltpu.async_copy(table.at[b[0]], b[3], b[4])

        def gwait(b):
            pltpu.make_async_copy(table.at[b[0]], b[3], b[4]).wait()

        def swait():
            pltpu.make_async_copy(sbuf, acc.at[dstS], ssem).wait()

        def scale_to_sbuf(b):
            rows = b[3]
            for g in range(CR // 16):
                w16 = b[2][pl.ds(g * 16, 16)]
                if scaled:
                    w16 = w16 * plsc.load_gather(rdv, [b[1][pl.ds(g * 16, 16)]])
                dstS[pl.ds(g * 16, 16)] = b[1][pl.ds(g * 16, 16)]
                for i in range(16):
                    e = g * 16 + i
                    wvi = lax.gather(
                        w16, jnp.full((16, 1), i, _i32), dn, (1,),
                        mode=lax.GatherScatterMode.PROMISE_IN_BOUNDS)
                    for c in range(_F // 16):
                        sbuf[e, pl.ds(c * 16, 16)] = rows[e, pl.ds(c * 16, 16)] * wvi

        def step(t, cur, last):
            gwait(cur)

            @pl.when(t >= 1)
            def _():
                swait()
            scale_to_sbuf(cur)
            pltpu.async_copy(sbuf, acc.at[dstS], ssem, add=True)
            if not last:
                load_idx(t + 2, cur)
                gstart(cur)

        load_idx(0, bufs[0])
        gstart(bufs[0])
        load_idx(1, bufs[1])
        gstart(bufs[1])

        def body(u, carry):
            t = u * 2
            step(t, bufs[0], False)
            step(t + 1, bufs[1], False)
            return carry

        lax.fori_loop(0, (T - 2) // 2, body, 0)
        step(T - 2, bufs[0], True)
        step(T - 1, bufs[1], True)
        swait()
        plsc.subcore_barrier()
        pltpu.sync_copy(acc.at[pl.ds(sid * _SEGM, _SEGM)],
                        out.at[cid, pl.ds(sid * _SEGM, _SEGM)])

    return k


def _vspec():
    return pl.BlockSpec((_BM, 1), lambda m: (m, 0))


def _mspec():
    return pl.BlockSpec((_BM, _F), lambda m: (m, 0))


def _wspec():
    return pl.BlockSpec((_F, _F), lambda m: (0, 0))


def _bspec():
    return pl.BlockSpec((1, _F), lambda m: (0, 0))


def _sspec():
    return pl.BlockSpec((1, 1), lambda m: (0, 0))


def _accspec(i):
    return pl.BlockSpec((None, _BM, _F), lambda m, i=i: (i, m, 0))


def _dot(a, b):
    return jnp.dot(a, b, preferred_element_type=_f32)


def _t1_body(x_ref, w_ref, dinv_ref, y_ref):
    y_ref[...] = _dot(x_ref[...], w_ref[...]) * dinv_ref[...]


def _t1(x, W, dinv_col):
    return pl.pallas_call(
        _t1_body,
        grid=(_GRID,),
        in_specs=[_mspec(), _wspec(), _vspec()],
        out_specs=_mspec(),
        out_shape=jax.ShapeDtypeStruct((_N, _F), _f32),
    )(x, W, dinv_col)


def _t2_body(a0, a1, y1, dinv, b1, wg, asv, adv, cem,
             h1o, xwo, aso, ado, selo):
    h1 = jnp.maximum((a0[...] + a1[...] + y1[...]) * dinv[...] + b1[...], 0.0)
    xw = _dot(h1, wg[...])
    a_s = _dot(xw, asv[...])
    a_d = _dot(xw, adv[...])
    asl = a_s + a_d + cem[0, 0]
    asl = jnp.maximum(asl, 0.2 * asl)
    h1o[...] = h1
    xwo[...] = xw
    aso[...] = a_s
    ado[...] = a_d
    selo[...] = asl


def _t2(acc, y1, dinv_col, b1, Wg, att_src_col, att_dst_col, cem):
    return pl.pallas_call(
        _t2_body,
        grid=(_GRID,),
        in_specs=[_accspec(0), _accspec(1), _mspec(), _vspec(), _bspec(),
                  _wspec(),
                  pl.BlockSpec((_F, 1), lambda m: (0, 0)),
                  pl.BlockSpec((_F, 1), lambda m: (0, 0)),
                  _sspec()],
        out_specs=[_mspec(), _mspec(), _vspec(), _vspec(), _vspec()],
        out_shape=[
            jax.ShapeDtypeStruct((_N, _F), _f32),
            jax.ShapeDtypeStruct((_N, _F), _f32),
            jax.ShapeDtypeStruct((_N, 1), _f32),
            jax.ShapeDtypeStruct((_N, 1), _f32),
            jax.ShapeDtypeStruct((_N, 1), _f32),
        ],
    )(acc, acc, y1, dinv_col, b1, Wg, att_src_col, att_dst_col, cem)


def _t3_body(d0, d1, asl, g, rdo, swo):
    exs = jnp.exp(asl[...] - g[0, 0])
    rd = 1.0 / (d0[...] + d1[...] + exs + 1e-16)
    rdo[...] = rd
    swo[...] = exs * rd


def _t3(d0, d1, aself_v, g):
    blk = pl.BlockSpec((_NPR, 128), lambda i: (0, 0))
    return pl.pallas_call(
        _t3_body,
        grid=(1,),
        in_specs=[blk, blk, blk, pl.BlockSpec((1, 1), lambda i: (0, 0))],
        out_specs=[blk, blk],
        out_shape=[jax.ShapeDtypeStruct((_NPR, 128), _f32),
                   jax.ShapeDtypeStruct((_NPR, 128), _f32)],
    )(d0, d1, aself_v, g)


def _t4_body(g0, g1, xw2, selfw, bg, w3, dinv, y3o):
    ha = jnp.maximum(g0[...] + g1[...] + selfw[...] * xw2[...] + bg[...], 0.0)
    y3o[...] = _dot(ha, w3[...]) * dinv[...]


def _t4(gacc, xw2, selfw_col, bg, W3, dinv_col):
    return pl.pallas_call(
        _t4_body,
        grid=(_GRID,),
        in_specs=[_accspec(0), _accspec(1), _mspec(), _vspec(), _bspec(),
                  _wspec(), _vspec()],
        out_specs=_mspec(),
        out_shape=jax.ShapeDtypeStruct((_N, _F), _f32),
    )(gacc, gacc, xw2, selfw_col, bg, W3, dinv_col)


def _t5_body(a0, a1, y3, dinv, b3, h1, qwt, kwt, vwt, qb, kb, vb, cs, feao):
    h2 = jnp.maximum((a0[...] + a1[...] + y3[...]) * dinv[...] + b3[...], 0.0)
    h1v = h1[...]
    Q0 = _dot(h1v, qwt[...]) + qb[...]
    Q1 = _dot(h2, qwt[...]) + qb[...]
    K0 = _dot(h1v, kwt[...]) + kb[...]
    K1 = _dot(h2, kwt[...]) + kb[...]
    V0 = _dot(h1v, vwt[...]) + vb[...]
    V1 = _dot(h2, vwt[...]) + vb[...]
    sc = 1.0 / sqrt(float(_F))
    s00 = jnp.sum(Q0 * K0, axis=1, keepdims=True) * sc
    s01 = jnp.sum(Q0 * K1, axis=1, keepdims=True) * sc
    s10 = jnp.sum(Q1 * K0, axis=1, keepdims=True) * sc
    s11 = jnp.sum(Q1 * K1, axis=1, keepdims=True) * sc
    c0 = cs[0, 0]
    c1 = cs[0, 1]
    cb = cs[0, 2]
    m0 = jnp.maximum(s00, s10)
    e00 = jnp.exp(s00 - m0)
    e10 = jnp.exp(s10 - m0)
    a00 = e00 / (e00 + e10)
    a10 = e10 / (e00 + e10)
    m1 = jnp.maximum(s01, s11)
    e01 = jnp.exp(s01 - m1)
    e11 = jnp.exp(s11 - m1)
    a01 = e01 / (e01 + e11)
    a11 = e11 / (e01 + e11)
    g0 = c0 * a00 + c1 * a10
    g1 = c0 * a01 + c1 * a11
    feao[...] = g0 * V0 + g1 * V1 + cb


def _t5(acc2, y3, dinv_col, b3, h1, qWT, kWT, vWT, qb, kb, vb, cs):
    return pl.pallas_call(
        _t5_body,
        grid=(_GRID,),
        in_specs=[_accspec(0), _accspec(1), _mspec(), _vspec(), _bspec(),
                  _mspec(), _wspec(), _wspec(), _wspec(),
                  _bspec(), _bspec(), _bspec(),
                  pl.BlockSpec((1, 3), lambda m: (0, 0))],
        out_specs=_mspec(),
        out_shape=jax.ShapeDtypeStruct((_N, _F), _f32),
    )(acc2, acc2, y3, dinv_col, b3, h1, qWT, kWT, vWT, qb, kb, vb, cs)


def _t6_body(a_ref, b_ref, o_ref):
    o_ref[...] = lax.dot_general(a_ref[...], b_ref[...],
                                 (((1,), (1,)), ((), ())),
                                 preferred_element_type=_f32)


_BT = 400


def _t6(A, B):
    return pl.pallas_call(
        _t6_body,
        grid=(_N // _BT,),
        in_specs=[pl.BlockSpec((_BT, _F), lambda i: (i, 0)),
                  pl.BlockSpec((_N, _F), lambda i: (0, 0))],
        out_specs=pl.BlockSpec((_BT, _N), lambda i: (i, 0)),
        out_shape=jax.ShapeDtypeStruct((_N, _N), _f32),
    )(A, B)


def _tdinv_body(d0, d1, o_ref):
    o_ref[...] = lax.rsqrt(d0[...] + d1[...] + 1.0)


def _tdinv(d0, d1):
    blk = pl.BlockSpec((_NPR, 128), lambda i: (0, 0))
    return pl.pallas_call(
        _tdinv_body,
        grid=(1,),
        in_specs=[blk, blk],
        out_specs=blk,
        out_shape=jax.ShapeDtypeStruct((_NPR, 128), _f32),
    )(d0, d1)


def _flat_pad(col):
    return jnp.pad(col[:, 0], (0, _NP - _N))


def _branch(x, edges, dm, p1, pg, p2, zv, zm):
    E = edges.shape[1]
    Ep = -(-E // (2 * _C * _NW)) * (2 * _C * _NW)
    npad = Ep - E
    pad_src = (jnp.arange(npad, dtype=_i32) * 97) % _N
    pad_dst = _N + (jnp.arange(npad, dtype=_i32) % (_NPM - _N))
    src = jnp.concatenate([edges[0].astype(_i32), pad_src])
    dst = jnp.concatenate([edges[1].astype(_i32), pad_dst])
    dm128 = jnp.reshape(dm, (_N * _N // 128, 128))

    ew, degp, sums = _make_ew_deg(Ep, E)(dm128, src, dst, zv)
    mean_ew = jnp.sum(sums) / E

    dinv_v = _tdinv(jnp.reshape(degp[0], (_NPR, 128)),
                    jnp.reshape(degp[1], (_NPR, 128)))
    dinv_flat = jnp.reshape(dinv_v, (_NP,))
    dinv_col = dinv_flat[:_N, None]

    y1 = _t1(x, p1["W"], dinv_col)
    acc = _make_rowscat(Ep, E, False)(y1, src, dst, ew, zm)
    ce = jnp.sum(pg["W_e"][0] * pg["att_edge"])
    h1, xw2, as_c, ad_c, asl_c = _t2(
        acc, y1, dinv_col, p1["b"][None, :], pg["W"],
        pg["att_src"][:, None], pg["att_dst"][:, None],
        jnp.reshape(ce * mean_ew, (1, 1)))
    asrc_flat = _flat_pad(as_c)
    adst_flat = _flat_pad(ad_c)
    cvec = jnp.broadcast_to(ce[None], (16,))

    alpha, amax = _make_alpha(Ep)(asrc_flat, adst_flat, ew, src, dst, cvec)
    g = jnp.maximum(jnp.max(amax), jnp.max(asl_c))
    gvec = jnp.broadcast_to(g[None], (16,))

    ex, denp = _make_denom(Ep)(alpha, dst, gvec, zv)
    rd_v, selfw_v = _t3(jnp.reshape(denp[0], (_NPR, 128)),
                        jnp.reshape(denp[1], (_NPR, 128)),
                        jnp.reshape(_flat_pad(asl_c), (_NPR, 128)),
                        jnp.reshape(g, (1, 1)))
    rd_flat = jnp.reshape(rd_v, (_NP,))
    selfw_col = jnp.reshape(selfw_v, (_NP,))[:_N, None]

    gacc = _make_rowscat(Ep, E, True)(xw2, src, dst, ex, rd_flat, zm)
    y3 = _t4(gacc, xw2, selfw_col, pg["b"][None, :], p2["W"], dinv_col)

    acc2 = _make_rowscat(Ep, E, False)(y3, src, dst, ew, zm)
    return h1, acc2, y3, dinv_col


def _att_branch(h1, acc2, y3, dinv_col, p2b, pa):
    cs = jnp.reshape(jnp.stack([pa["cW"][0, 0, 0], pa["cW"][0, 1, 0],
                                pa["cb"][0]]), (1, 3))
    return _t5(acc2, y3, dinv_col, p2b[None, :], h1,
               pa["qW"].T, pa["kW"].T, pa["vW"].T,
               pa["qb"][None, :], pa["kb"][None, :], pa["vb"][None, :], cs)


def kernel(ll_edges, ll_data_matrix, dd_edges, dd_data_matrix, params):
    kx = jax.random.key(1)
    x_lnc = jax.random.normal(jax.random.fold_in(kx, 0), (_N, _F), dtype=_f32)
    x_dis = jax.random.normal(jax.random.fold_in(kx, 1), (_N, _F), dtype=_f32)
    zv = jnp.zeros((_NP,), _f32)
    zm = jnp.zeros((_NPM, _F), _f32)

    l_parts = _branch(x_lnc, ll_edges, ll_data_matrix,
                      params["gcn_lnc1"], params["gat_lnc1"],
                      params["gcn_lnc2"], zv, zm)
    d_parts = _branch(x_dis, dd_edges, dd_data_matrix,
                      params["gcn_dis1"], params["gat_dis1"],
                      params["gcn_dis2"], zv, zm)
    lnc = _att_branch(*l_parts, params["gcn_lnc2"]["b"], params["att_lnc"])
    dis = _att_branch(*d_parts, params["gcn_dis2"]["b"], params["att_dis"])
    return _t6(lnc, dis), lnc, dis
